# baseline (speedup 1.0000x reference)
import functools

import jax
import jax.numpy as jnp
from jax import lax
from jax.experimental import pallas as pl
from jax.experimental.pallas import tpu as pltpu
from jax.experimental.pallas import tpu_sc as plsc

N = 2048
E = 16384
F = 128
TBW = 256
AGW = 256
NTILES = 32
EPT = E // NTILES
CH = 128
NCH = EPT // CH
ROWS_PER_TILE = N // 16


def _leaky(x):
    return jnp.maximum(x, 0.2 * x)


def _sigmoid(x):
    return 1.0 / (1.0 + jnp.exp(-x))


def _tanh(x):
    return 1.0 - 2.0 / (jnp.exp(2.0 * x) + 1.0)


def _elu(x):
    return jnp.where(x > 0, x, jnp.exp(jnp.minimum(x, 0.0)) - 1.0)


def _erf(x):
    s = jnp.sign(x)
    ax = jnp.abs(x)
    t = 1.0 / (1.0 + 0.3275911 * ax)
    poly = ((((1.061405429 * t - 1.453152027) * t + 1.421413741) * t
             - 0.284496736) * t + 0.254829592) * t
    return s * (1.0 - poly * jnp.exp(-ax * ax))


def _gelu(x):
    return 0.5 * x * (1.0 + _erf(x * 0.7071067811865476))


def _dott(x, w):
    return lax.dot_general(x, w, (((1,), (1,)), ((), ())),
                           preferred_element_type=jnp.float32)


def _gru(x, h, wih, whh, bih, bhh):
    gi = _dott(x, wih) + bih
    gh = _dott(h, whh) + bhh
    r = _sigmoid(gi[:, 0:F] + gh[:, 0:F])
    z = _sigmoid(gi[:, F:2 * F] + gh[:, F:2 * F])
    cand = _tanh(gi[:, 2 * F:3 * F] + r * gh[:, 2 * F:3 * F])
    return (1.0 - z) * cand + z * h


def _pad_ones(n, w):
    col = lax.broadcasted_iota(jnp.int32, (n, w), 1)
    return jnp.where(col == 0, 1.0, 0.0).astype(jnp.float32)


def _tc1_body(atom, ea, w1, b1, wfc2, bfc2, attn_w, ab, wat, bat,
              newx_o, tbl_o, et_o, alpha_o, a2_o):
    at = atom[...]
    nx = _leaky(_dott(at, w1[...]) + b1[...])
    newx_o[...] = nx
    wfc2v = wfc2[...]
    a = _dott(at, wfc2v[:, 0:F])
    nft1 = _dott(nx, wat[...]) + bat[...]
    tbl_o[...] = jnp.concatenate([a, nft1], axis=1)
    et_o[...] = _dott(ea[...], wfc2v[:, F:F + 16]) + bfc2[...]
    aw = attn_w[...]
    alpha = jnp.sum(nx * aw[:, 0:F], axis=1, keepdims=True) + ab[...]
    alpha_o[...] = alpha.reshape(N // F, F)
    a2_o[...] = aw[:, F:2 * F]


def _tc1(atom, ea, w1, b1, wfc2, bfc2, attn_w, ab, wat, bat):
    return pl.pallas_call(
        _tc1_body,
        out_shape=(
            jax.ShapeDtypeStruct((N, F), jnp.float32),
            jax.ShapeDtypeStruct((N, TBW), jnp.float32),
            jax.ShapeDtypeStruct((E, F), jnp.float32),
            jax.ShapeDtypeStruct((N // F, F), jnp.float32),
            jax.ShapeDtypeStruct((1, F), jnp.float32),
        ),
    )(atom, ea, w1, b1, wfc2, bfc2, attn_w, ab, wat, bat)


def _tc2_body(acc, newx, wih, whh, bih, bhh, wfc22, bfc22, w2attn, b21,
              out1_o, tbl2_o, b1_o, b2_o):
    s = acc[0:N] + acc[N:2 * N]
    t = s[:, F:2 * F] / s[:, 0:1]
    ctx = _elu(t)
    nx = newx[...]
    out1 = _gru(ctx, nx, wih[...], whh[...], bih[...], bhh[...])
    out1_o[...] = out1
    tbl2_o[...] = _dott(out1, wfc22[...]) + bfc22[...]
    aw = w2attn[...]
    b1 = jnp.sum(out1 * aw[:, 0:F], axis=1, keepdims=True) + b21[...]
    b2 = jnp.sum(out1 * aw[:, F:2 * F], axis=1, keepdims=True)
    b1_o[...] = b1.reshape(N // F, F)
    b2_o[...] = b2.reshape(N // F, F)


def _tc2(acc, newx, wih, whh, bih, bhh, wfc22, bfc22, w2attn, b21):
    return pl.pallas_call(
        _tc2_body,
        out_shape=(
            jax.ShapeDtypeStruct((N, F), jnp.float32),
            jax.ShapeDtypeStruct((N, F), jnp.float32),
            jax.ShapeDtypeStruct((N // F, F), jnp.float32),
            jax.ShapeDtypeStruct((N // F, F), jnp.float32),
        ),
    )(acc, newx, wih, whh, bih, bhh, wfc22, bfc22, w2attn, b21)


def _tc3_body(acc, out1, wih, whh, bih, bhh, lin, linb,
              out_o, allt_o, avg_o):
    s = acc[0:N] + acc[N:2 * N]
    t = s[:, F:2 * F] / s[:, 0:1]
    ctx = _elu(t)
    o1 = out1[...]
    out2 = _gru(ctx, o1, wih[...], whh[...], bih[...], bhh[...])
    allt_o[0] = o1
    allt_o[1] = out2
    avg = (o1 + out2) * 0.5
    avg_o[...] = avg
    pre = _dott(avg, lin[...]) + linb[...]
    out_o[...] = _gelu(pre)


def _tc3(acc, out1, wih, whh, bih, bhh, lin, linb):
    return pl.pallas_call(
        _tc3_body,
        out_shape=(
            jax.ShapeDtypeStruct((N, F), jnp.float32),
            jax.ShapeDtypeStruct((2, N, F), jnp.float32),
            jax.ShapeDtypeStruct((N, F), jnp.float32),
        ),
    )(acc, out1, wih, whh, bih, bhh, lin, linb)


_MESH = plsc.VectorSubcoreMesh(core_axis_name="c", subcore_axis_name="s")
_SC_PARAMS = pltpu.CompilerParams(use_tc_tiling_on_sc=False,
                                  needs_layout_passes=False)


def _zero_shared_slice(sbuf, acc_sh, sid):
    zv = jnp.zeros((16,), jnp.float32)

    @plsc.parallel_loop(0, CH, unroll=4)
    def _(r):
        for k in range(AGW // 16):
            sbuf[r, pl.ds(k * 16, 16)] = zv
    pltpu.sync_copy(sbuf, acc_sh.at[pl.ds(sid * ROWS_PER_TILE, ROWS_PER_TILE)])


def _scale_rows(gbuf, sbuf, wbuf, col_off, e0):
    @plsc.parallel_loop(0, CH, unroll=4)
    def _(r):
        wsp = plsc.load_gather(wbuf, [jnp.full((16,), r, jnp.int32)])
        sbuf[r, pl.ds(0, 16)] = wsp * e0
        for k in range(F // 16):
            sbuf[r, pl.ds(F + k * 16, 16)] = gbuf[r, pl.ds(col_off + k * 16, 16)] * wsp


def _sc1(tbl, et, alpha, a2, src2, dst2):
    @functools.partial(
        pl.kernel,
        mesh=_MESH,
        compiler_params=_SC_PARAMS,
        out_type=jax.ShapeDtypeStruct((2 * N, AGW), jnp.float32),
        scratch_types=[
            pltpu.VMEM((N // F, F), jnp.float32),
            pltpu.VMEM((1, F), jnp.float32),
            pltpu.VMEM((NCH, CH), jnp.int32),
            pltpu.VMEM((NCH, CH), jnp.int32),
            pltpu.VMEM((CH, TBW), jnp.float32),
            pltpu.VMEM((CH, F), jnp.float32),
            pltpu.VMEM((CH, AGW), jnp.float32),
            pltpu.VMEM((CH,), jnp.float32),
            pltpu.VMEM((CH, 17), jnp.float32),
            pltpu.VMEM_SHARED((N, AGW), jnp.float32),
            pltpu.SemaphoreType.DMA,
        ],
    )
    def k(tbl_h, et_h, alpha_h, a2_h, src_h, dst_h, out_h,
          alpha_v, a2_v, src_v, dst_v, gbuf, ebuf, sbuf, wbuf, ptmp, acc_sh, sem):
        cid = lax.axis_index("c")
        sid = lax.axis_index("s")
        wid = cid * 16 + sid
        ebase = wid * EPT
        pltpu.sync_copy(alpha_h, alpha_v)
        pltpu.sync_copy(a2_h, a2_v)
        pltpu.sync_copy(src_h.at[pl.ds(wid * NCH, NCH)], src_v)
        pltpu.sync_copy(dst_h.at[pl.ds(wid * NCH, NCH)], dst_v)
        _zero_shared_slice(sbuf, acc_sh, sid)
        plsc.subcore_barrier()

        iot = lax.iota(jnp.int32, 16)
        zero16 = jnp.zeros((16,), jnp.float32)
        e0 = jnp.where(iot == 0, 1.0, 0.0).astype(jnp.float32)
        a2blk = [a2_v[0, pl.ds(k * 16, 16)] for k in range(F // 16)]
        for c in range(NCH):
            pltpu.sync_copy(et_h.at[pl.ds(ebase + c * CH, CH)], ebuf)
            pltpu.async_copy(tbl_h.at[src_v.at[c]], gbuf, sem).wait()

            @plsc.parallel_loop(0, CH, unroll=4)
            def _(r):
                acc = zero16
                for k in range(F // 16):
                    a = gbuf[r, pl.ds(k * 16, 16)]
                    e = ebuf[r, pl.ds(k * 16, 16)]
                    acc = acc + _leaky(a + e) * a2blk[k]
                ptmp[r, pl.ds(0, 16)] = acc

            for g in range(CH // 16):
                rows = iot + g * 16
                tot = zero16
                for k in range(16):
                    tot = tot + plsc.load_gather(
                        ptmp, [rows, jnp.full((16,), k, jnp.int32)])
                dstv = dst_v[c, pl.ds(g * 16, 16)]
                ad = plsc.load_gather(
                    alpha_v,
                    [lax.shift_right_logical(dstv, 7), dstv & 127])
                wbuf[pl.ds(g * 16, 16)] = jnp.exp(_leaky(tot + ad))
            _scale_rows(gbuf, sbuf, wbuf, F, e0)
            pltpu.sync_copy(sbuf, acc_sh.at[dst_v.at[c]], add=True)

        plsc.subcore_barrier()
        pltpu.sync_copy(
            acc_sh.at[pl.ds(sid * ROWS_PER_TILE, ROWS_PER_TILE)],
            out_h.at[pl.ds(cid * N + sid * ROWS_PER_TILE, ROWS_PER_TILE)])

    return k(tbl, et, alpha, a2, src2, dst2)


def _sc2(tbl2, b1, b2, src2, dst2):
    @functools.partial(
        pl.kernel,
        mesh=_MESH,
        compiler_params=_SC_PARAMS,
        out_type=jax.ShapeDtypeStruct((2 * N, AGW), jnp.float32),
        scratch_types=[
            pltpu.VMEM((N // F, F), jnp.float32),
            pltpu.VMEM((N // F, F), jnp.float32),
            pltpu.VMEM((NCH, CH), jnp.int32),
            pltpu.VMEM((NCH, CH), jnp.int32),
            pltpu.VMEM((CH, F), jnp.float32),
            pltpu.VMEM((CH, AGW), jnp.float32),
            pltpu.VMEM((CH,), jnp.float32),
            pltpu.VMEM_SHARED((N, AGW), jnp.float32),
            pltpu.SemaphoreType.DMA,
        ],
    )
    def k(tbl_h, b1_h, b2_h, src_h, dst_h, out_h,
          b1_v, b2_v, src_v, dst_v, gbuf, sbuf, wbuf, acc_sh, sem):
        cid = lax.axis_index("c")
        sid = lax.axis_index("s")
        wid = cid * 16 + sid
        pltpu.sync_copy(b1_h, b1_v)
        pltpu.sync_copy(b2_h, b2_v)
        pltpu.sync_copy(src_h.at[pl.ds(wid * NCH, NCH)], src_v)
        pltpu.sync_copy(dst_h.at[pl.ds(wid * NCH, NCH)], dst_v)
        _zero_shared_slice(sbuf, acc_sh, sid)
        plsc.subcore_barrier()

        iot = lax.iota(jnp.int32, 16)
        e0 = jnp.where(iot == 0, 1.0, 0.0).astype(jnp.float32)
        for c in range(NCH):
            pltpu.async_copy(tbl_h.at[src_v.at[c]], gbuf, sem).wait()
            for g in range(CH // 16):
                dstv = dst_v[c, pl.ds(g * 16, 16)]
                srcv = src_v[c, pl.ds(g * 16, 16)]
                bd = plsc.load_gather(
                    b1_v, [lax.shift_right_logical(dstv, 7), dstv & 127])
                bs = plsc.load_gather(
                    b2_v, [lax.shift_right_logical(srcv, 7), srcv & 127])
                wbuf[pl.ds(g * 16, 16)] = jnp.exp(_leaky(bd + bs))
            _scale_rows(gbuf, sbuf, wbuf, 0, e0)
            pltpu.sync_copy(sbuf, acc_sh.at[dst_v.at[c]], add=True)

        plsc.subcore_barrier()
        pltpu.sync_copy(
            acc_sh.at[pl.ds(sid * ROWS_PER_TILE, ROWS_PER_TILE)],
            out_h.at[pl.ds(cid * N + sid * ROWS_PER_TILE, ROWS_PER_TILE)])

    return k(tbl2, b1, b2, src2, dst2)


def kernel(atom_features, edge_index, edge_attr,
           v1_fc1_w, v1_fc1_b, v1_fc2_w, v1_fc2_b,
           v1_attn_w, v1_attn_b, v1_attend_w, v1_attend_b,
           v1_gru_wih, v1_gru_whh, v1_gru_bih, v1_gru_bhh,
           v2_fc1_w, v2_fc1_b, v2_fc2_w, v2_fc2_b,
           v2_gru_wih, v2_gru_whh, v2_gru_bih, v2_gru_bhh,
           lin_w, lin_b):
    src2 = edge_index[0].reshape(E // CH, CH)
    dst2 = edge_index[1].reshape(E // CH, CH)

    newx, tbl, et, alpha, a2 = _tc1(
        atom_features, edge_attr,
        v1_fc1_w, v1_fc1_b.reshape(1, F),
        v1_fc2_w, v1_fc2_b.reshape(1, F),
        v1_attn_w, v1_attn_b.reshape(1, 1),
        v1_attend_w, v1_attend_b.reshape(1, F))

    acc1 = _sc1(tbl, et, alpha, a2, src2, dst2)

    out1, tbl2, b1, b2 = _tc2(
        acc1, newx,
        v1_gru_wih, v1_gru_whh,
        v1_gru_bih.reshape(1, 3 * F), v1_gru_bhh.reshape(1, 3 * F),
        v2_fc2_w, v2_fc2_b.reshape(1, F),
        v2_fc1_w, v2_fc1_b.reshape(1, 1))

    acc2 = _sc2(tbl2, b1, b2, src2, dst2)

    output, all_t, avg = _tc3(
        acc2, out1,
        v2_gru_wih, v2_gru_whh,
        v2_gru_bih.reshape(1, 3 * F), v2_gru_bhh.reshape(1, 3 * F),
        lin_w, lin_b.reshape(1, F))

    return (output, all_t, newx, avg)


# speedup vs baseline: 1.1732x; 1.1732x over previous
import functools

import jax
import jax.numpy as jnp
from jax import lax
from jax.experimental import pallas as pl
from jax.experimental.pallas import tpu as pltpu
from jax.experimental.pallas import tpu_sc as plsc

N = 2048
E = 16384
F = 128
NTILES = 32
EPT = E // NTILES
CH = 128
NCH = EPT // CH
ROWS_PER_TILE = N // 16


def _leaky(x):
    return jnp.maximum(x, 0.2 * x)


def _sigmoid(x):
    return 1.0 / (1.0 + jnp.exp(-x))


def _tanh(x):
    return 1.0 - 2.0 / (jnp.exp(2.0 * x) + 1.0)


def _elu(x):
    return jnp.where(x > 0, x, jnp.exp(jnp.minimum(x, 0.0)) - 1.0)


def _erf(x):
    s = jnp.sign(x)
    ax = jnp.abs(x)
    t = 1.0 / (1.0 + 0.3275911 * ax)
    poly = ((((1.061405429 * t - 1.453152027) * t + 1.421413741) * t
             - 0.284496736) * t + 0.254829592) * t
    return s * (1.0 - poly * jnp.exp(-ax * ax))


def _gelu(x):
    return 0.5 * x * (1.0 + _erf(x * 0.7071067811865476))


def _dott(x, w):
    return lax.dot_general(x, w, (((1,), (1,)), ((), ())),
                           preferred_element_type=jnp.float32)


def _gru(x, h, wih, whh, bih, bhh):
    gi = _dott(x, wih) + bih
    gh = _dott(h, whh) + bhh
    r = _sigmoid(gi[:, 0:F] + gh[:, 0:F])
    z = _sigmoid(gi[:, F:2 * F] + gh[:, F:2 * F])
    cand = _tanh(gi[:, 2 * F:3 * F] + r * gh[:, 2 * F:3 * F])
    return (1.0 - z) * cand + z * h


def _tc1_body(atom, ea_t, w1, b1, wfc2, bfc2, attn_w, ab, wat, bat,
              newx_o, tbla_o, tblb_o, et_o, alpha_o, a2_o):
    at = atom[...]
    nx = _leaky(_dott(at, w1[...]) + b1[...])
    newx_o[...] = nx
    wfc2v = wfc2[...]
    tbla_o[...] = _dott(at, wfc2v[:, 0:F])
    tblb_o[...] = _dott(nx, wat[...]) + bat[...]
    et_o[...] = lax.dot_general(
        ea_t[...], wfc2v[:, F:F + 16], (((0,), (1,)), ((), ())),
        preferred_element_type=jnp.float32) + bfc2[...]
    aw = attn_w[...]
    alpha = jnp.sum(nx * aw[:, 0:F], axis=1, keepdims=True) + ab[...]
    alpha_o[...] = alpha.reshape(N // F, F)
    a2_o[...] = aw[:, F:2 * F]


def _tc1(atom, ea_t, w1, b1, wfc2, bfc2, attn_w, ab, wat, bat):
    return pl.pallas_call(
        _tc1_body,
        out_shape=(
            jax.ShapeDtypeStruct((N, F), jnp.float32),
            jax.ShapeDtypeStruct((N, F), jnp.float32),
            jax.ShapeDtypeStruct((N, F), jnp.float32),
            jax.ShapeDtypeStruct((E, F), jnp.float32),
            jax.ShapeDtypeStruct((N // F, F), jnp.float32),
            jax.ShapeDtypeStruct((1, F), jnp.float32),
        ),
    )(atom, ea_t, w1, b1, wfc2, bfc2, attn_w, ab, wat, bat)


def _tc2_body(accv, accd, newx, wih, whh, bih, bhh, wfc22, bfc22, w2attn, b21,
              out1_o, tbl2_o, b1_o, b2_o):
    t = (accv[0:N] + accv[N:2 * N]) / (accd[0:N] + accd[N:2 * N])
    ctx = _elu(t)
    nx = newx[...]
    out1 = _gru(ctx, nx, wih[...], whh[...], bih[...], bhh[...])
    out1_o[...] = out1
    tbl2_o[...] = _dott(out1, wfc22[...]) + bfc22[...]
    aw = w2attn[...]
    b1 = jnp.sum(out1 * aw[:, 0:F], axis=1, keepdims=True) + b21[...]
    b2 = jnp.sum(out1 * aw[:, F:2 * F], axis=1, keepdims=True)
    b1_o[...] = b1.reshape(N // F, F)
    b2_o[...] = b2.reshape(N // F, F)


def _tc2(accv, accd, newx, wih, whh, bih, bhh, wfc22, bfc22, w2attn, b21):
    return pl.pallas_call(
        _tc2_body,
        out_shape=(
            jax.ShapeDtypeStruct((N, F), jnp.float32),
            jax.ShapeDtypeStruct((N, F), jnp.float32),
            jax.ShapeDtypeStruct((N // F, F), jnp.float32),
            jax.ShapeDtypeStruct((N // F, F), jnp.float32),
        ),
    )(accv, accd, newx, wih, whh, bih, bhh, wfc22, bfc22, w2attn, b21)


def _tc3_body(accv, accd, out1, wih, whh, bih, bhh, lin, linb,
              out_o, allt_o, avg_o):
    t = (accv[0:N] + accv[N:2 * N]) / (accd[0:N] + accd[N:2 * N])
    ctx = _elu(t)
    o1 = out1[...]
    out2 = _gru(ctx, o1, wih[...], whh[...], bih[...], bhh[...])
    allt_o[0] = o1
    allt_o[1] = out2
    avg = (o1 + out2) * 0.5
    avg_o[...] = avg
    pre = _dott(avg, lin[...]) + linb[...]
    out_o[...] = _gelu(pre)


def _tc3(accv, accd, out1, wih, whh, bih, bhh, lin, linb):
    return pl.pallas_call(
        _tc3_body,
        out_shape=(
            jax.ShapeDtypeStruct((N, F), jnp.float32),
            jax.ShapeDtypeStruct((2, N, F), jnp.float32),
            jax.ShapeDtypeStruct((N, F), jnp.float32),
        ),
    )(accv, accd, out1, wih, whh, bih, bhh, lin, linb)


_MESH = plsc.VectorSubcoreMesh(core_axis_name="c", subcore_axis_name="s")
_SC_PARAMS = pltpu.CompilerParams(use_tc_tiling_on_sc=False,
                                  needs_layout_passes=False)


def _zero_shared(sbuf, accv_sh, accd_sh, sid):
    zv = jnp.zeros((16,), jnp.float32)

    @plsc.parallel_loop(0, CH, unroll=4)
    def _(r):
        for k in range(F // 16):
            sbuf[r, pl.ds(k * 16, 16)] = zv
    sl = pl.ds(sid * ROWS_PER_TILE, ROWS_PER_TILE)
    pltpu.sync_copy(sbuf, accv_sh.at[sl])
    pltpu.sync_copy(sbuf, accd_sh.at[sl])


def _scale_rows(gbuf, sbufv, sbufd, wbuf):
    @plsc.parallel_loop(0, CH, unroll=4)
    def _(r):
        wsp = plsc.load_gather(wbuf, [jnp.full((16,), r, jnp.int32)])
        for k in range(F // 16):
            sbufv[r, pl.ds(k * 16, 16)] = gbuf[r, pl.ds(k * 16, 16)] * wsp
            sbufd[r, pl.ds(k * 16, 16)] = wsp


def _emit_partials(accv_sh, accd_sh, outv_h, outd_h, cid, sid):
    sl = pl.ds(sid * ROWS_PER_TILE, ROWS_PER_TILE)
    osl = pl.ds(cid * N + sid * ROWS_PER_TILE, ROWS_PER_TILE)
    pltpu.sync_copy(accv_sh.at[sl], outv_h.at[osl])
    pltpu.sync_copy(accd_sh.at[sl], outd_h.at[osl])


def _sc1(tbla, tblb, et, alpha, a2, src2, dst2):
    @functools.partial(
        pl.kernel,
        mesh=_MESH,
        compiler_params=_SC_PARAMS,
        out_type=(
            jax.ShapeDtypeStruct((2 * N, F), jnp.float32),
            jax.ShapeDtypeStruct((2 * N, F), jnp.float32),
        ),
        scratch_types=[
            pltpu.VMEM((N // F, F), jnp.float32),
            pltpu.VMEM((1, F), jnp.float32),
            pltpu.VMEM((NCH, CH), jnp.int32),
            pltpu.VMEM((NCH, CH), jnp.int32),
            pltpu.VMEM((CH, F), jnp.float32),
            pltpu.VMEM((CH, F), jnp.float32),
            pltpu.VMEM((CH, F), jnp.float32),
            pltpu.VMEM((CH, F), jnp.float32),
            pltpu.VMEM((CH, F), jnp.float32),
            pltpu.VMEM((CH,), jnp.float32),
            pltpu.VMEM((CH, 17), jnp.float32),
            pltpu.VMEM_SHARED((N, F), jnp.float32),
            pltpu.VMEM_SHARED((N, F), jnp.float32),
            pltpu.SemaphoreType.DMA,
            pltpu.SemaphoreType.DMA,
        ],
    )
    def k(tbla_h, tblb_h, et_h, alpha_h, a2_h, src_h, dst_h, outv_h, outd_h,
          alpha_v, a2_v, src_v, dst_v, gbufa, gbufb, ebuf, sbufv, sbufd,
          wbuf, ptmp, accv_sh, accd_sh, sema, semb):
        cid = lax.axis_index("c")
        sid = lax.axis_index("s")
        wid = cid * 16 + sid
        ebase = wid * EPT
        pltpu.sync_copy(alpha_h, alpha_v)
        pltpu.sync_copy(a2_h, a2_v)
        pltpu.sync_copy(src_h.at[pl.ds(wid * NCH, NCH)], src_v)
        pltpu.sync_copy(dst_h.at[pl.ds(wid * NCH, NCH)], dst_v)
        _zero_shared(sbufv, accv_sh, accd_sh, sid)
        plsc.subcore_barrier()

        iot = lax.iota(jnp.int32, 16)
        zero16 = jnp.zeros((16,), jnp.float32)
        a2blk = [a2_v[0, pl.ds(k * 16, 16)] for k in range(F // 16)]
        for c in range(NCH):
            pltpu.sync_copy(et_h.at[pl.ds(ebase + c * CH, CH)], ebuf)
            cpa = pltpu.async_copy(tbla_h.at[src_v.at[c]], gbufa, sema)
            cpb = pltpu.async_copy(tblb_h.at[src_v.at[c]], gbufb, semb)
            cpa.wait()
            cpb.wait()

            @plsc.parallel_loop(0, CH, unroll=4)
            def _(r):
                acc = zero16
                for k in range(F // 16):
                    a = gbufa[r, pl.ds(k * 16, 16)]
                    e = ebuf[r, pl.ds(k * 16, 16)]
                    acc = acc + _leaky(a + e) * a2blk[k]
                ptmp[r, pl.ds(0, 16)] = acc

            for g in range(CH // 16):
                rows = iot + g * 16
                tot = zero16
                for k in range(16):
                    tot = tot + plsc.load_gather(
                        ptmp, [rows, jnp.full((16,), k, jnp.int32)])
                dstv = dst_v[c, pl.ds(g * 16, 16)]
                ad = plsc.load_gather(
                    alpha_v,
                    [lax.shift_right_logical(dstv, 7), dstv & 127])
                wbuf[pl.ds(g * 16, 16)] = jnp.exp(_leaky(tot + ad))

            _scale_rows(gbufb, sbufv, sbufd, wbuf)
            pltpu.sync_copy(sbufv, accv_sh.at[dst_v.at[c]], add=True)
            pltpu.sync_copy(sbufd, accd_sh.at[dst_v.at[c]], add=True)

        plsc.subcore_barrier()
        _emit_partials(accv_sh, accd_sh, outv_h, outd_h, cid, sid)

    return k(tbla, tblb, et, alpha, a2, src2, dst2)


def _sc2(tbl2, b1, b2, src2, dst2):
    @functools.partial(
        pl.kernel,
        mesh=_MESH,
        compiler_params=_SC_PARAMS,
        out_type=(
            jax.ShapeDtypeStruct((2 * N, F), jnp.float32),
            jax.ShapeDtypeStruct((2 * N, F), jnp.float32),
        ),
        scratch_types=[
            pltpu.VMEM((N // F, F), jnp.float32),
            pltpu.VMEM((N // F, F), jnp.float32),
            pltpu.VMEM((NCH, CH), jnp.int32),
            pltpu.VMEM((NCH, CH), jnp.int32),
            pltpu.VMEM((CH, F), jnp.float32),
            pltpu.VMEM((CH, F), jnp.float32),
            pltpu.VMEM((CH, F), jnp.float32),
            pltpu.VMEM((CH,), jnp.float32),
            pltpu.VMEM_SHARED((N, F), jnp.float32),
            pltpu.VMEM_SHARED((N, F), jnp.float32),
            pltpu.SemaphoreType.DMA,
        ],
    )
    def k(tbl_h, b1_h, b2_h, src_h, dst_h, outv_h, outd_h,
          b1_v, b2_v, src_v, dst_v, gbuf, sbufv, sbufd, wbuf,
          accv_sh, accd_sh, sem):
        cid = lax.axis_index("c")
        sid = lax.axis_index("s")
        wid = cid * 16 + sid
        pltpu.sync_copy(b1_h, b1_v)
        pltpu.sync_copy(b2_h, b2_v)
        pltpu.sync_copy(src_h.at[pl.ds(wid * NCH, NCH)], src_v)
        pltpu.sync_copy(dst_h.at[pl.ds(wid * NCH, NCH)], dst_v)
        _zero_shared(sbufv, accv_sh, accd_sh, sid)
        plsc.subcore_barrier()

        for c in range(NCH):
            pltpu.async_copy(tbl_h.at[src_v.at[c]], gbuf, sem).wait()
            for g in range(CH // 16):
                dstv = dst_v[c, pl.ds(g * 16, 16)]
                srcv = src_v[c, pl.ds(g * 16, 16)]
                bd = plsc.load_gather(
                    b1_v, [lax.shift_right_logical(dstv, 7), dstv & 127])
                bs = plsc.load_gather(
                    b2_v, [lax.shift_right_logical(srcv, 7), srcv & 127])
                wbuf[pl.ds(g * 16, 16)] = jnp.exp(_leaky(bd + bs))
            _scale_rows(gbuf, sbufv, sbufd, wbuf)
            pltpu.sync_copy(sbufv, accv_sh.at[dst_v.at[c]], add=True)
            pltpu.sync_copy(sbufd, accd_sh.at[dst_v.at[c]], add=True)

        plsc.subcore_barrier()
        _emit_partials(accv_sh, accd_sh, outv_h, outd_h, cid, sid)

    return k(tbl2, b1, b2, src2, dst2)


def kernel(atom_features, edge_index, edge_attr,
           v1_fc1_w, v1_fc1_b, v1_fc2_w, v1_fc2_b,
           v1_attn_w, v1_attn_b, v1_attend_w, v1_attend_b,
           v1_gru_wih, v1_gru_whh, v1_gru_bih, v1_gru_bhh,
           v2_fc1_w, v2_fc1_b, v2_fc2_w, v2_fc2_b,
           v2_gru_wih, v2_gru_whh, v2_gru_bih, v2_gru_bhh,
           lin_w, lin_b):
    src2 = edge_index[0].reshape(E // CH, CH)
    dst2 = edge_index[1].reshape(E // CH, CH)

    newx, tbla, tblb, et, alpha, a2 = _tc1(
        atom_features, edge_attr.T,
        v1_fc1_w, v1_fc1_b.reshape(1, F),
        v1_fc2_w, v1_fc2_b.reshape(1, F),
        v1_attn_w, v1_attn_b.reshape(1, 1),
        v1_attend_w, v1_attend_b.reshape(1, F))

    accv1, accd1 = _sc1(tbla, tblb, et, alpha, a2, src2, dst2)

    out1, tbl2, b1, b2 = _tc2(
        accv1, accd1, newx,
        v1_gru_wih, v1_gru_whh,
        v1_gru_bih.reshape(1, 3 * F), v1_gru_bhh.reshape(1, 3 * F),
        v2_fc2_w, v2_fc2_b.reshape(1, F),
        v2_fc1_w, v2_fc1_b.reshape(1, 1))

    accv2, accd2 = _sc2(tbl2, b1, b2, src2, dst2)

    output, all_t, avg = _tc3(
        accv2, accd2, out1,
        v2_gru_wih, v2_gru_whh,
        v2_gru_bih.reshape(1, 3 * F), v2_gru_bhh.reshape(1, 3 * F),
        lin_w, lin_b.reshape(1, F))

    return (output, all_t, newx, avg)


# speedup vs baseline: 1.2276x; 1.0463x over previous
import functools

import jax
import jax.numpy as jnp
from jax import lax
from jax.experimental import pallas as pl
from jax.experimental.pallas import tpu as pltpu
from jax.experimental.pallas import tpu_sc as plsc

N = 2048
E = 16384
F = 128
NTILES = 32
EPT = E // NTILES
CH = 64
NCH = EPT // CH
ROWS_PER_TILE = N // 16


def _leaky(x):
    return jnp.maximum(x, 0.2 * x)


def _sigmoid(x):
    return 1.0 / (1.0 + jnp.exp(-x))


def _tanh(x):
    return 1.0 - 2.0 / (jnp.exp(2.0 * x) + 1.0)


def _elu(x):
    return jnp.where(x > 0, x, jnp.exp(jnp.minimum(x, 0.0)) - 1.0)


def _erf(x):
    s = jnp.sign(x)
    ax = jnp.abs(x)
    t = 1.0 / (1.0 + 0.3275911 * ax)
    poly = ((((1.061405429 * t - 1.453152027) * t + 1.421413741) * t
             - 0.284496736) * t + 0.254829592) * t
    return s * (1.0 - poly * jnp.exp(-ax * ax))


def _gelu(x):
    return 0.5 * x * (1.0 + _erf(x * 0.7071067811865476))


def _dott(x, w):
    return lax.dot_general(x, w, (((1,), (1,)), ((), ())),
                           preferred_element_type=jnp.float32)


def _gru(x, h, wih, whh, bih, bhh):
    gi = _dott(x, wih) + bih
    gh = _dott(h, whh) + bhh
    r = _sigmoid(gi[:, 0:F] + gh[:, 0:F])
    z = _sigmoid(gi[:, F:2 * F] + gh[:, F:2 * F])
    cand = _tanh(gi[:, 2 * F:3 * F] + r * gh[:, 2 * F:3 * F])
    return (1.0 - z) * cand + z * h


def _tc1_body(atom, ea_t, w1, b1, wfc2, bfc2, attn_w, ab, wat, bat,
              newx_o, tbla_o, tblb_o, et_o, alpha_o, a2_o):
    at = atom[...]
    nx = _leaky(_dott(at, w1[...]) + b1[...])
    newx_o[...] = nx
    wfc2v = wfc2[...]
    tbla_o[...] = _dott(at, wfc2v[:, 0:F])
    tblb_o[...] = _dott(nx, wat[...]) + bat[...]
    et_o[...] = lax.dot_general(
        ea_t[...], wfc2v[:, F:F + 16], (((0,), (1,)), ((), ())),
        preferred_element_type=jnp.float32) + bfc2[...]
    aw = attn_w[...]
    alpha = jnp.sum(nx * aw[:, 0:F], axis=1, keepdims=True) + ab[...]
    alpha_o[...] = alpha.reshape(N // F, F)
    a2_o[...] = aw[:, F:2 * F]


def _tc1(atom, ea_t, w1, b1, wfc2, bfc2, attn_w, ab, wat, bat):
    return pl.pallas_call(
        _tc1_body,
        out_shape=(
            jax.ShapeDtypeStruct((N, F), jnp.float32),
            jax.ShapeDtypeStruct((N, F), jnp.float32),
            jax.ShapeDtypeStruct((N, F), jnp.float32),
            jax.ShapeDtypeStruct((E, F), jnp.float32),
            jax.ShapeDtypeStruct((N // F, F), jnp.float32),
            jax.ShapeDtypeStruct((1, F), jnp.float32),
        ),
    )(atom, ea_t, w1, b1, wfc2, bfc2, attn_w, ab, wat, bat)


def _tc2_body(accv, accd, newx, wih, whh, bih, bhh, wfc22, bfc22, w2attn, b21,
              out1_o, tbl2_o, b1_o, b2_o):
    t = (accv[0:N] + accv[N:2 * N]) / (accd[0:N] + accd[N:2 * N])
    ctx = _elu(t)
    nx = newx[...]
    out1 = _gru(ctx, nx, wih[...], whh[...], bih[...], bhh[...])
    out1_o[...] = out1
    tbl2_o[...] = _dott(out1, wfc22[...]) + bfc22[...]
    aw = w2attn[...]
    b1 = jnp.sum(out1 * aw[:, 0:F], axis=1, keepdims=True) + b21[...]
    b2 = jnp.sum(out1 * aw[:, F:2 * F], axis=1, keepdims=True)
    b1_o[...] = b1.reshape(N // F, F)
    b2_o[...] = b2.reshape(N // F, F)


def _tc2(accv, accd, newx, wih, whh, bih, bhh, wfc22, bfc22, w2attn, b21):
    return pl.pallas_call(
        _tc2_body,
        out_shape=(
            jax.ShapeDtypeStruct((N, F), jnp.float32),
            jax.ShapeDtypeStruct((N, F), jnp.float32),
            jax.ShapeDtypeStruct((N // F, F), jnp.float32),
            jax.ShapeDtypeStruct((N // F, F), jnp.float32),
        ),
    )(accv, accd, newx, wih, whh, bih, bhh, wfc22, bfc22, w2attn, b21)


def _tc3_body(accv, accd, out1, wih, whh, bih, bhh, lin, linb,
              out_o, allt_o, avg_o):
    t = (accv[0:N] + accv[N:2 * N]) / (accd[0:N] + accd[N:2 * N])
    ctx = _elu(t)
    o1 = out1[...]
    out2 = _gru(ctx, o1, wih[...], whh[...], bih[...], bhh[...])
    allt_o[0] = o1
    allt_o[1] = out2
    avg = (o1 + out2) * 0.5
    avg_o[...] = avg
    pre = _dott(avg, lin[...]) + linb[...]
    out_o[...] = _gelu(pre)


def _tc3(accv, accd, out1, wih, whh, bih, bhh, lin, linb):
    return pl.pallas_call(
        _tc3_body,
        out_shape=(
            jax.ShapeDtypeStruct((N, F), jnp.float32),
            jax.ShapeDtypeStruct((2, N, F), jnp.float32),
            jax.ShapeDtypeStruct((N, F), jnp.float32),
        ),
    )(accv, accd, out1, wih, whh, bih, bhh, lin, linb)


_MESH = plsc.VectorSubcoreMesh(core_axis_name="c", subcore_axis_name="s")
_SC_PARAMS = pltpu.CompilerParams(use_tc_tiling_on_sc=False,
                                  needs_layout_passes=False)


def _zero_shared(sbuf, accv_sh, accd_sh, sid):
    zv = jnp.zeros((16,), jnp.float32)

    @plsc.parallel_loop(0, CH, unroll=4)
    def _(r):
        for k in range(F // 16):
            sbuf[r, pl.ds(k * 16, 16)] = zv
    for h in range(ROWS_PER_TILE // CH):
        sl = pl.ds(sid * ROWS_PER_TILE + h * CH, CH)
        pltpu.sync_copy(sbuf, accv_sh.at[sl])
        pltpu.sync_copy(sbuf, accd_sh.at[sl])


def _scale_rows(gbuf, sbufv, sbufd, wbuf):
    @plsc.parallel_loop(0, CH, unroll=4)
    def _(r):
        wsp = plsc.load_gather(wbuf, [jnp.full((16,), r, jnp.int32)])
        for k in range(F // 16):
            sbufv[r, pl.ds(k * 16, 16)] = gbuf[r, pl.ds(k * 16, 16)] * wsp
            sbufd[r, pl.ds(k * 16, 16)] = wsp


def _emit_partials(accv_sh, accd_sh, outv_h, outd_h, cid, sid):
    sl = pl.ds(sid * ROWS_PER_TILE, ROWS_PER_TILE)
    osl = pl.ds(cid * N + sid * ROWS_PER_TILE, ROWS_PER_TILE)
    pltpu.sync_copy(accv_sh.at[sl], outv_h.at[osl])
    pltpu.sync_copy(accd_sh.at[sl], outd_h.at[osl])


def _sc1(tbla, tblb, et, alpha, a2, src2, dst2):
    @functools.partial(
        pl.kernel,
        mesh=_MESH,
        compiler_params=_SC_PARAMS,
        out_type=(
            jax.ShapeDtypeStruct((2 * N, F), jnp.float32),
            jax.ShapeDtypeStruct((2 * N, F), jnp.float32),
        ),
        scratch_types=[
            pltpu.VMEM((N // F, F), jnp.float32),
            pltpu.VMEM((1, F), jnp.float32),
            pltpu.VMEM((NCH, CH), jnp.int32),
            pltpu.VMEM((NCH, CH), jnp.int32),
            pltpu.VMEM((2, CH, F), jnp.float32),
            pltpu.VMEM((2, CH, F), jnp.float32),
            pltpu.VMEM((CH, F), jnp.float32),
            pltpu.VMEM((CH, F), jnp.float32),
            pltpu.VMEM((CH, F), jnp.float32),
            pltpu.VMEM((CH,), jnp.float32),
            pltpu.VMEM((CH, 17), jnp.float32),
            pltpu.VMEM_SHARED((N, F), jnp.float32),
            pltpu.VMEM_SHARED((N, F), jnp.float32),
            [pltpu.SemaphoreType.DMA] * 4,
        ],
    )
    def k(tbla_h, tblb_h, et_h, alpha_h, a2_h, src_h, dst_h, outv_h, outd_h,
          alpha_v, a2_v, src_v, dst_v, gbufa, gbufb, ebuf, sbufv, sbufd,
          wbuf, ptmp, accv_sh, accd_sh, sems):
        cid = lax.axis_index("c")
        sid = lax.axis_index("s")
        wid = cid * 16 + sid
        ebase = wid * EPT
        pltpu.sync_copy(alpha_h, alpha_v)
        pltpu.sync_copy(a2_h, a2_v)
        pltpu.sync_copy(src_h.at[pl.ds(wid * NCH, NCH)], src_v)
        pltpu.sync_copy(dst_h.at[pl.ds(wid * NCH, NCH)], dst_v)
        _zero_shared(sbufv, accv_sh, accd_sh, sid)
        plsc.subcore_barrier()

        iot = lax.iota(jnp.int32, 16)
        zero16 = jnp.zeros((16,), jnp.float32)
        a2blk = [a2_v[0, pl.ds(k * 16, 16)] for k in range(F // 16)]

        def issue(c):
            s = c % 2
            cpa = pltpu.async_copy(tbla_h.at[src_v.at[c]], gbufa.at[s],
                                   sems[2 * s])
            cpb = pltpu.async_copy(tblb_h.at[src_v.at[c]], gbufb.at[s],
                                   sems[2 * s + 1])
            return cpa, cpb

        pend = issue(0)
        for c in range(NCH):
            s = c % 2
            pltpu.sync_copy(et_h.at[pl.ds(ebase + c * CH, CH)], ebuf)
            pend[0].wait()
            pend[1].wait()
            if c + 1 < NCH:
                pend = issue(c + 1)

            @plsc.parallel_loop(0, CH, unroll=4)
            def _(r):
                acc = zero16
                for k in range(F // 16):
                    a = gbufa[s, r, pl.ds(k * 16, 16)]
                    e = ebuf[r, pl.ds(k * 16, 16)]
                    acc = acc + _leaky(a + e) * a2blk[k]
                ptmp[r, pl.ds(0, 16)] = acc

            for g in range(CH // 16):
                rows = iot + g * 16
                tot = zero16
                for k in range(16):
                    tot = tot + plsc.load_gather(
                        ptmp, [rows, jnp.full((16,), k, jnp.int32)])
                dstv = dst_v[c, pl.ds(g * 16, 16)]
                ad = plsc.load_gather(
                    alpha_v,
                    [lax.shift_right_logical(dstv, 7), dstv & 127])
                wbuf[pl.ds(g * 16, 16)] = jnp.exp(_leaky(tot + ad))

            _scale_rows(gbufb.at[s], sbufv, sbufd, wbuf)
            pltpu.sync_copy(sbufv, accv_sh.at[dst_v.at[c]], add=True)
            pltpu.sync_copy(sbufd, accd_sh.at[dst_v.at[c]], add=True)

        plsc.subcore_barrier()
        _emit_partials(accv_sh, accd_sh, outv_h, outd_h, cid, sid)

    return k(tbla, tblb, et, alpha, a2, src2, dst2)


def _sc2(tbl2, b1, b2, src2, dst2):
    @functools.partial(
        pl.kernel,
        mesh=_MESH,
        compiler_params=_SC_PARAMS,
        out_type=(
            jax.ShapeDtypeStruct((2 * N, F), jnp.float32),
            jax.ShapeDtypeStruct((2 * N, F), jnp.float32),
        ),
        scratch_types=[
            pltpu.VMEM((N // F, F), jnp.float32),
            pltpu.VMEM((N // F, F), jnp.float32),
            pltpu.VMEM((NCH, CH), jnp.int32),
            pltpu.VMEM((NCH, CH), jnp.int32),
            pltpu.VMEM((2, CH, F), jnp.float32),
            pltpu.VMEM((CH, F), jnp.float32),
            pltpu.VMEM((CH, F), jnp.float32),
            pltpu.VMEM((CH,), jnp.float32),
            pltpu.VMEM_SHARED((N, F), jnp.float32),
            pltpu.VMEM_SHARED((N, F), jnp.float32),
            [pltpu.SemaphoreType.DMA] * 2,
        ],
    )
    def k(tbl_h, b1_h, b2_h, src_h, dst_h, outv_h, outd_h,
          b1_v, b2_v, src_v, dst_v, gbuf, sbufv, sbufd, wbuf,
          accv_sh, accd_sh, sems):
        cid = lax.axis_index("c")
        sid = lax.axis_index("s")
        wid = cid * 16 + sid
        pltpu.sync_copy(b1_h, b1_v)
        pltpu.sync_copy(b2_h, b2_v)
        pltpu.sync_copy(src_h.at[pl.ds(wid * NCH, NCH)], src_v)
        pltpu.sync_copy(dst_h.at[pl.ds(wid * NCH, NCH)], dst_v)
        _zero_shared(sbufv, accv_sh, accd_sh, sid)
        plsc.subcore_barrier()

        def issue(c):
            s = c % 2
            return pltpu.async_copy(tbl_h.at[src_v.at[c]], gbuf.at[s], sems[s])

        pend = issue(0)
        for c in range(NCH):
            s = c % 2
            pend.wait()
            if c + 1 < NCH:
                pend = issue(c + 1)
            for g in range(CH // 16):
                dstv = dst_v[c, pl.ds(g * 16, 16)]
                srcv = src_v[c, pl.ds(g * 16, 16)]
                bd = plsc.load_gather(
                    b1_v, [lax.shift_right_logical(dstv, 7), dstv & 127])
                bs = plsc.load_gather(
                    b2_v, [lax.shift_right_logical(srcv, 7), srcv & 127])
                wbuf[pl.ds(g * 16, 16)] = jnp.exp(_leaky(bd + bs))
            _scale_rows(gbuf.at[s], sbufv, sbufd, wbuf)
            pltpu.sync_copy(sbufv, accv_sh.at[dst_v.at[c]], add=True)
            pltpu.sync_copy(sbufd, accd_sh.at[dst_v.at[c]], add=True)

        plsc.subcore_barrier()
        _emit_partials(accv_sh, accd_sh, outv_h, outd_h, cid, sid)

    return k(tbl2, b1, b2, src2, dst2)


def kernel(atom_features, edge_index, edge_attr,
           v1_fc1_w, v1_fc1_b, v1_fc2_w, v1_fc2_b,
           v1_attn_w, v1_attn_b, v1_attend_w, v1_attend_b,
           v1_gru_wih, v1_gru_whh, v1_gru_bih, v1_gru_bhh,
           v2_fc1_w, v2_fc1_b, v2_fc2_w, v2_fc2_b,
           v2_gru_wih, v2_gru_whh, v2_gru_bih, v2_gru_bhh,
           lin_w, lin_b):
    src2 = edge_index[0].reshape(E // CH, CH)
    dst2 = edge_index[1].reshape(E // CH, CH)

    newx, tbla, tblb, et, alpha, a2 = _tc1(
        atom_features, edge_attr.T,
        v1_fc1_w, v1_fc1_b.reshape(1, F),
        v1_fc2_w, v1_fc2_b.reshape(1, F),
        v1_attn_w, v1_attn_b.reshape(1, 1),
        v1_attend_w, v1_attend_b.reshape(1, F))

    accv1, accd1 = _sc1(tbla, tblb, et, alpha, a2, src2, dst2)

    out1, tbl2, b1, b2 = _tc2(
        accv1, accd1, newx,
        v1_gru_wih, v1_gru_whh,
        v1_gru_bih.reshape(1, 3 * F), v1_gru_bhh.reshape(1, 3 * F),
        v2_fc2_w, v2_fc2_b.reshape(1, F),
        v2_fc1_w, v2_fc1_b.reshape(1, 1))

    accv2, accd2 = _sc2(tbl2, b1, b2, src2, dst2)

    output, all_t, avg = _tc3(
        accv2, accd2, out1,
        v2_gru_wih, v2_gru_whh,
        v2_gru_bih.reshape(1, 3 * F), v2_gru_bhh.reshape(1, 3 * F),
        lin_w, lin_b.reshape(1, F))

    return (output, all_t, newx, avg)


# speedup vs baseline: 1.3874x; 1.1302x over previous
import functools

import jax
import jax.numpy as jnp
from jax import lax
from jax.experimental import pallas as pl
from jax.experimental.pallas import tpu as pltpu
from jax.experimental.pallas import tpu_sc as plsc

N = 2048
E = 16384
F = 128
NTILES = 32
EPT = E // NTILES
CH = 64
NCH = EPT // CH
ROWS_PER_TILE = N // 16


def _leaky(x):
    return jnp.maximum(x, 0.2 * x)


def _sigmoid(x):
    return 1.0 / (1.0 + jnp.exp(-x))


def _tanh(x):
    return 1.0 - 2.0 / (jnp.exp(2.0 * x) + 1.0)


def _elu(x):
    return jnp.where(x > 0, x, jnp.exp(jnp.minimum(x, 0.0)) - 1.0)


def _erf(x):
    s = jnp.sign(x)
    ax = jnp.abs(x)
    t = 1.0 / (1.0 + 0.3275911 * ax)
    poly = ((((1.061405429 * t - 1.453152027) * t + 1.421413741) * t
             - 0.284496736) * t + 0.254829592) * t
    return s * (1.0 - poly * jnp.exp(-ax * ax))


def _gelu(x):
    return 0.5 * x * (1.0 + _erf(x * 0.7071067811865476))


def _dott(x, w):
    return lax.dot_general(x, w, (((1,), (1,)), ((), ())),
                           preferred_element_type=jnp.float32)


def _gru(x, h, wih, whh, bih, bhh):
    gi = _dott(x, wih) + bih
    gh = _dott(h, whh) + bhh
    r = _sigmoid(gi[:, 0:F] + gh[:, 0:F])
    z = _sigmoid(gi[:, F:2 * F] + gh[:, F:2 * F])
    cand = _tanh(gi[:, 2 * F:3 * F] + r * gh[:, 2 * F:3 * F])
    return (1.0 - z) * cand + z * h


def _tc1_body(atom, ea_t, w1, b1, wfc2, bfc2, attn_w, ab, wat, bat,
              newx_o, tbla_o, tblb_o, et_o, alpha_o, a2_o):
    at = atom[...]
    nx = _leaky(_dott(at, w1[...]) + b1[...])
    newx_o[...] = nx
    wfc2v = wfc2[...]
    tbla_o[...] = _dott(at, wfc2v[:, 0:F])
    tblb_o[...] = _dott(nx, wat[...]) + bat[...]
    et_o[...] = lax.dot_general(
        ea_t[...], wfc2v[:, F:F + 16], (((0,), (1,)), ((), ())),
        preferred_element_type=jnp.float32) + bfc2[...]
    aw = attn_w[...]
    alpha = jnp.sum(nx * aw[:, 0:F], axis=1, keepdims=True) + ab[...]
    alpha_o[...] = alpha.reshape(N // F, F)
    a2_o[...] = aw[:, F:2 * F]


def _tc1(atom, ea_t, w1, b1, wfc2, bfc2, attn_w, ab, wat, bat):
    return pl.pallas_call(
        _tc1_body,
        out_shape=(
            jax.ShapeDtypeStruct((N, F), jnp.float32),
            jax.ShapeDtypeStruct((N, F), jnp.float32),
            jax.ShapeDtypeStruct((N, F), jnp.float32),
            jax.ShapeDtypeStruct((E, F), jnp.float32),
            jax.ShapeDtypeStruct((N // F, F), jnp.float32),
            jax.ShapeDtypeStruct((1, F), jnp.float32),
        ),
    )(atom, ea_t, w1, b1, wfc2, bfc2, attn_w, ab, wat, bat)


def _tc2_body(accv, accd, newx, wih, whh, bih, bhh, wfc22, bfc22, w2attn, b21,
              out1_o, tbl2_o, b1_o, b2_o):
    t = (accv[0:N] + accv[N:2 * N]) / (accd[0:N] + accd[N:2 * N])
    ctx = _elu(t)
    nx = newx[...]
    out1 = _gru(ctx, nx, wih[...], whh[...], bih[...], bhh[...])
    out1_o[...] = out1
    tbl2_o[...] = _dott(out1, wfc22[...]) + bfc22[...]
    aw = w2attn[...]
    b1 = jnp.sum(out1 * aw[:, 0:F], axis=1, keepdims=True) + b21[...]
    b2 = jnp.sum(out1 * aw[:, F:2 * F], axis=1, keepdims=True)
    b1_o[...] = b1.reshape(N // F, F)
    b2_o[...] = b2.reshape(N // F, F)


def _tc2(accv, accd, newx, wih, whh, bih, bhh, wfc22, bfc22, w2attn, b21):
    return pl.pallas_call(
        _tc2_body,
        out_shape=(
            jax.ShapeDtypeStruct((N, F), jnp.float32),
            jax.ShapeDtypeStruct((N, F), jnp.float32),
            jax.ShapeDtypeStruct((N // F, F), jnp.float32),
            jax.ShapeDtypeStruct((N // F, F), jnp.float32),
        ),
    )(accv, accd, newx, wih, whh, bih, bhh, wfc22, bfc22, w2attn, b21)


def _tc3_body(accv, accd, out1, wih, whh, bih, bhh, lin, linb,
              out_o, allt_o, avg_o):
    t = (accv[0:N] + accv[N:2 * N]) / (accd[0:N] + accd[N:2 * N])
    ctx = _elu(t)
    o1 = out1[...]
    out2 = _gru(ctx, o1, wih[...], whh[...], bih[...], bhh[...])
    allt_o[0] = o1
    allt_o[1] = out2
    avg = (o1 + out2) * 0.5
    avg_o[...] = avg
    pre = _dott(avg, lin[...]) + linb[...]
    out_o[...] = _gelu(pre)


def _tc3(accv, accd, out1, wih, whh, bih, bhh, lin, linb):
    return pl.pallas_call(
        _tc3_body,
        out_shape=(
            jax.ShapeDtypeStruct((N, F), jnp.float32),
            jax.ShapeDtypeStruct((2, N, F), jnp.float32),
            jax.ShapeDtypeStruct((N, F), jnp.float32),
        ),
    )(accv, accd, out1, wih, whh, bih, bhh, lin, linb)


_MESH = plsc.VectorSubcoreMesh(core_axis_name="c", subcore_axis_name="s")
_SC_PARAMS = pltpu.CompilerParams(use_tc_tiling_on_sc=False,
                                  needs_layout_passes=False)


def _zero_shared(sbuf, accv_sh, accd_sh, sid):
    zv = jnp.zeros((16,), jnp.float32)

    @plsc.parallel_loop(0, CH, unroll=4)
    def _(r):
        for k in range(F // 16):
            sbuf[r, pl.ds(k * 16, 16)] = zv
    for h in range(ROWS_PER_TILE // CH):
        sl = pl.ds(sid * ROWS_PER_TILE + h * CH, CH)
        pltpu.sync_copy(sbuf, accv_sh.at[sl])
        pltpu.sync_copy(sbuf, accd_sh.at[sl])


def _scale_rows(gbuf, sbufv, sbufd, wbuf):
    @plsc.parallel_loop(0, CH, unroll=4)
    def _(r):
        wsp = plsc.load_gather(wbuf, [jnp.full((16,), r, jnp.int32)])
        for k in range(F // 16):
            sbufv[r, pl.ds(k * 16, 16)] = gbuf[r, pl.ds(k * 16, 16)] * wsp
            sbufd[r, pl.ds(k * 16, 16)] = wsp


def _emit_partials(accv_sh, accd_sh, outv_h, outd_h, cid, sid):
    sl = pl.ds(sid * ROWS_PER_TILE, ROWS_PER_TILE)
    osl = pl.ds(cid * N + sid * ROWS_PER_TILE, ROWS_PER_TILE)
    pltpu.sync_copy(accv_sh.at[sl], outv_h.at[osl])
    pltpu.sync_copy(accd_sh.at[sl], outd_h.at[osl])


def _sc1(tbla, tblb, et, alpha, a2, src2, dst2):
    @functools.partial(
        pl.kernel,
        mesh=_MESH,
        compiler_params=_SC_PARAMS,
        out_type=(
            jax.ShapeDtypeStruct((2 * N, F), jnp.float32),
            jax.ShapeDtypeStruct((2 * N, F), jnp.float32),
        ),
        scratch_types=[
            pltpu.VMEM((N // F, F), jnp.float32),
            pltpu.VMEM((1, F), jnp.float32),
            pltpu.VMEM((NCH, CH), jnp.int32),
            pltpu.VMEM((NCH, CH), jnp.int32),
            pltpu.VMEM((2, CH, F), jnp.float32),
            pltpu.VMEM((2, CH, F), jnp.float32),
            pltpu.VMEM((2, CH, F), jnp.float32),
            pltpu.VMEM((CH, F), jnp.float32),
            pltpu.VMEM((CH, F), jnp.float32),
            pltpu.VMEM((CH,), jnp.float32),
            pltpu.VMEM((CH, 17), jnp.float32),
            pltpu.VMEM_SHARED((N, F), jnp.float32),
            pltpu.VMEM_SHARED((N, F), jnp.float32),
            [pltpu.SemaphoreType.DMA] * 6,
        ],
    )
    def k(tbla_h, tblb_h, et_h, alpha_h, a2_h, src_h, dst_h, outv_h, outd_h,
          alpha_v, a2_v, src_v, dst_v, gbufa, gbufb, ebuf, sbufv, sbufd,
          wbuf, ptmp, accv_sh, accd_sh, sems):
        cid = lax.axis_index("c")
        sid = lax.axis_index("s")
        wid = cid * 16 + sid
        ebase = wid * EPT
        pre = [pltpu.async_copy(alpha_h, alpha_v, sems[0]),
               pltpu.async_copy(a2_h, a2_v, sems[1]),
               pltpu.async_copy(src_h.at[pl.ds(wid * NCH, NCH)], src_v, sems[2]),
               pltpu.async_copy(dst_h.at[pl.ds(wid * NCH, NCH)], dst_v, sems[3])]
        _zero_shared(sbufv, accv_sh, accd_sh, sid)
        for cp in pre:
            cp.wait()
        plsc.subcore_barrier()

        iot = lax.iota(jnp.int32, 16)
        zero16 = jnp.zeros((16,), jnp.float32)
        a2blk = [a2_v[0, pl.ds(k * 16, 16)] for k in range(F // 16)]

        def issue(c):
            s = c % 2
            cpa = pltpu.async_copy(tbla_h.at[src_v.at[c]], gbufa.at[s],
                                   sems[2 * s])
            cpb = pltpu.async_copy(tblb_h.at[src_v.at[c]], gbufb.at[s],
                                   sems[2 * s + 1])
            cpe = pltpu.async_copy(et_h.at[pl.ds(ebase + c * CH, CH)],
                                   ebuf.at[s], sems[4 + s])
            return cpa, cpb, cpe

        pend = issue(0)
        for c in range(NCH):
            s = c % 2
            for cp in pend:
                cp.wait()
            if c + 1 < NCH:
                pend = issue(c + 1)

            @plsc.parallel_loop(0, CH, unroll=4)
            def _(r):
                acc = zero16
                for k in range(F // 16):
                    a = gbufa[s, r, pl.ds(k * 16, 16)]
                    e = ebuf[s, r, pl.ds(k * 16, 16)]
                    acc = acc + _leaky(a + e) * a2blk[k]
                ptmp[r, pl.ds(0, 16)] = acc

            for g in range(CH // 16):
                rows = iot + g * 16
                tot = zero16
                for k in range(16):
                    tot = tot + plsc.load_gather(
                        ptmp, [rows, jnp.full((16,), k, jnp.int32)])
                dstv = dst_v[c, pl.ds(g * 16, 16)]
                ad = plsc.load_gather(
                    alpha_v,
                    [lax.shift_right_logical(dstv, 7), dstv & 127])
                wbuf[pl.ds(g * 16, 16)] = jnp.exp(_leaky(tot + ad))

            _scale_rows(gbufb.at[s], sbufv, sbufd, wbuf)
            pltpu.sync_copy(sbufv, accv_sh.at[dst_v.at[c]], add=True)
            pltpu.sync_copy(sbufd, accd_sh.at[dst_v.at[c]], add=True)

        plsc.subcore_barrier()
        _emit_partials(accv_sh, accd_sh, outv_h, outd_h, cid, sid)

    return k(tbla, tblb, et, alpha, a2, src2, dst2)


def _sc2(tbl2, b1, b2, src2, dst2):
    @functools.partial(
        pl.kernel,
        mesh=_MESH,
        compiler_params=_SC_PARAMS,
        out_type=(
            jax.ShapeDtypeStruct((2 * N, F), jnp.float32),
            jax.ShapeDtypeStruct((2 * N, F), jnp.float32),
        ),
        scratch_types=[
            pltpu.VMEM((N // F, F), jnp.float32),
            pltpu.VMEM((N // F, F), jnp.float32),
            pltpu.VMEM((NCH, CH), jnp.int32),
            pltpu.VMEM((NCH, CH), jnp.int32),
            pltpu.VMEM((2, CH, F), jnp.float32),
            pltpu.VMEM((CH, F), jnp.float32),
            pltpu.VMEM((CH, F), jnp.float32),
            pltpu.VMEM((CH,), jnp.float32),
            pltpu.VMEM_SHARED((N, F), jnp.float32),
            pltpu.VMEM_SHARED((N, F), jnp.float32),
            [pltpu.SemaphoreType.DMA] * 4,
        ],
    )
    def k(tbl_h, b1_h, b2_h, src_h, dst_h, outv_h, outd_h,
          b1_v, b2_v, src_v, dst_v, gbuf, sbufv, sbufd, wbuf,
          accv_sh, accd_sh, sems):
        cid = lax.axis_index("c")
        sid = lax.axis_index("s")
        wid = cid * 16 + sid
        pre = [pltpu.async_copy(b1_h, b1_v, sems[0]),
               pltpu.async_copy(b2_h, b2_v, sems[1]),
               pltpu.async_copy(src_h.at[pl.ds(wid * NCH, NCH)], src_v, sems[2]),
               pltpu.async_copy(dst_h.at[pl.ds(wid * NCH, NCH)], dst_v, sems[3])]
        _zero_shared(sbufv, accv_sh, accd_sh, sid)
        for cp in pre:
            cp.wait()
        plsc.subcore_barrier()

        def issue(c):
            s = c % 2
            return pltpu.async_copy(tbl_h.at[src_v.at[c]], gbuf.at[s], sems[s])

        pend = issue(0)
        for c in range(NCH):
            s = c % 2
            pend.wait()
            if c + 1 < NCH:
                pend = issue(c + 1)
            for g in range(CH // 16):
                dstv = dst_v[c, pl.ds(g * 16, 16)]
                srcv = src_v[c, pl.ds(g * 16, 16)]
                bd = plsc.load_gather(
                    b1_v, [lax.shift_right_logical(dstv, 7), dstv & 127])
                bs = plsc.load_gather(
                    b2_v, [lax.shift_right_logical(srcv, 7), srcv & 127])
                wbuf[pl.ds(g * 16, 16)] = jnp.exp(_leaky(bd + bs))
            _scale_rows(gbuf.at[s], sbufv, sbufd, wbuf)
            pltpu.sync_copy(sbufv, accv_sh.at[dst_v.at[c]], add=True)
            pltpu.sync_copy(sbufd, accd_sh.at[dst_v.at[c]], add=True)

        plsc.subcore_barrier()
        _emit_partials(accv_sh, accd_sh, outv_h, outd_h, cid, sid)

    return k(tbl2, b1, b2, src2, dst2)


def kernel(atom_features, edge_index, edge_attr,
           v1_fc1_w, v1_fc1_b, v1_fc2_w, v1_fc2_b,
           v1_attn_w, v1_attn_b, v1_attend_w, v1_attend_b,
           v1_gru_wih, v1_gru_whh, v1_gru_bih, v1_gru_bhh,
           v2_fc1_w, v2_fc1_b, v2_fc2_w, v2_fc2_b,
           v2_gru_wih, v2_gru_whh, v2_gru_bih, v2_gru_bhh,
           lin_w, lin_b):
    src2 = edge_index[0].reshape(E // CH, CH)
    dst2 = edge_index[1].reshape(E // CH, CH)

    newx, tbla, tblb, et, alpha, a2 = _tc1(
        atom_features, edge_attr.T,
        v1_fc1_w, v1_fc1_b.reshape(1, F),
        v1_fc2_w, v1_fc2_b.reshape(1, F),
        v1_attn_w, v1_attn_b.reshape(1, 1),
        v1_attend_w, v1_attend_b.reshape(1, F))

    accv1, accd1 = _sc1(tbla, tblb, et, alpha, a2, src2, dst2)

    out1, tbl2, b1, b2 = _tc2(
        accv1, accd1, newx,
        v1_gru_wih, v1_gru_whh,
        v1_gru_bih.reshape(1, 3 * F), v1_gru_bhh.reshape(1, 3 * F),
        v2_fc2_w, v2_fc2_b.reshape(1, F),
        v2_fc1_w, v2_fc1_b.reshape(1, 1))

    accv2, accd2 = _sc2(tbl2, b1, b2, src2, dst2)

    output, all_t, avg = _tc3(
        accv2, accd2, out1,
        v2_gru_wih, v2_gru_whh,
        v2_gru_bih.reshape(1, 3 * F), v2_gru_bhh.reshape(1, 3 * F),
        lin_w, lin_b.reshape(1, F))

    return (output, all_t, newx, avg)


# speedup vs baseline: 1.4031x; 1.0113x over previous
import functools

import jax
import jax.numpy as jnp
from jax import lax
from jax.experimental import pallas as pl
from jax.experimental.pallas import tpu as pltpu
from jax.experimental.pallas import tpu_sc as plsc

N = 2048
E = 16384
F = 128
NTILES = 32
EPT = E // NTILES
CH = 64
NCH = EPT // CH
ROWS_PER_TILE = N // 16


def _leaky(x):
    return jnp.maximum(x, 0.2 * x)


def _sigmoid(x):
    return 1.0 / (1.0 + jnp.exp(-x))


def _tanh(x):
    return 1.0 - 2.0 / (jnp.exp(2.0 * x) + 1.0)


def _elu(x):
    return jnp.where(x > 0, x, jnp.exp(jnp.minimum(x, 0.0)) - 1.0)


def _erf(x):
    s = jnp.sign(x)
    ax = jnp.abs(x)
    t = 1.0 / (1.0 + 0.3275911 * ax)
    poly = ((((1.061405429 * t - 1.453152027) * t + 1.421413741) * t
             - 0.284496736) * t + 0.254829592) * t
    return s * (1.0 - poly * jnp.exp(-ax * ax))


def _gelu(x):
    return 0.5 * x * (1.0 + _erf(x * 0.7071067811865476))


def _dott(x, w):
    return lax.dot_general(x, w, (((1,), (1,)), ((), ())),
                           preferred_element_type=jnp.float32)


def _gru(x, h, wih, whh, bih, bhh):
    gi = _dott(x, wih) + bih
    gh = _dott(h, whh) + bhh
    r = _sigmoid(gi[:, 0:F] + gh[:, 0:F])
    z = _sigmoid(gi[:, F:2 * F] + gh[:, F:2 * F])
    cand = _tanh(gi[:, 2 * F:3 * F] + r * gh[:, 2 * F:3 * F])
    return (1.0 - z) * cand + z * h


def _tc1_body(atom, ea_t, w1, b1, wfc2, bfc2, attn_w, ab, wat, bat,
              newx_o, tbla_o, tblb_o, et_o, alpha_o, a2_o):
    at = atom[...]
    nx = _leaky(_dott(at, w1[...]) + b1[...])
    newx_o[...] = nx
    wfc2v = wfc2[...]
    tbla_o[...] = _dott(at, wfc2v[:, 0:F])
    tblb_o[...] = _dott(nx, wat[...]) + bat[...]
    et_o[...] = lax.dot_general(
        ea_t[...], wfc2v[:, F:F + 16], (((0,), (1,)), ((), ())),
        preferred_element_type=jnp.float32) + bfc2[...]
    aw = attn_w[...]
    alpha = jnp.sum(nx * aw[:, 0:F], axis=1, keepdims=True) + ab[...]
    alpha_o[...] = alpha.reshape(N // F, F)
    a2_o[...] = aw[:, F:2 * F]


def _tc1(atom, ea_t, w1, b1, wfc2, bfc2, attn_w, ab, wat, bat):
    return pl.pallas_call(
        _tc1_body,
        out_shape=(
            jax.ShapeDtypeStruct((N, F), jnp.float32),
            jax.ShapeDtypeStruct((N, F), jnp.float32),
            jax.ShapeDtypeStruct((N, F), jnp.float32),
            jax.ShapeDtypeStruct((E, F), jnp.float32),
            jax.ShapeDtypeStruct((N // F, F), jnp.float32),
            jax.ShapeDtypeStruct((1, F), jnp.float32),
        ),
    )(atom, ea_t, w1, b1, wfc2, bfc2, attn_w, ab, wat, bat)


_GB = 512
_NG = N // _GB


def _tc2_body(accv_lo, accv_hi, accd_lo, accd_hi, newx, wih, whh, bih, bhh,
              wfc22, bfc22, w2attn, b21,
              out1_o, tbl2_o, b1_o, b2_o):
    t = (accv_lo[...] + accv_hi[...]) / (accd_lo[...] + accd_hi[...])
    ctx = _elu(t)
    nx = newx[...]
    out1 = _gru(ctx, nx, wih[...], whh[...], bih[...], bhh[...])
    out1_o[...] = out1
    tbl2_o[...] = _dott(out1, wfc22[...]) + bfc22[...]
    aw = w2attn[...]
    b1 = jnp.sum(out1 * aw[:, 0:F], axis=1, keepdims=True) + b21[...]
    b2 = jnp.sum(out1 * aw[:, F:2 * F], axis=1, keepdims=True)
    pid = pl.program_id(0)
    b1_o[pl.ds(pid * (_GB // F), _GB // F), :] = b1.reshape(_GB // F, F)
    b2_o[pl.ds(pid * (_GB // F), _GB // F), :] = b2.reshape(_GB // F, F)


def _rows(i):
    return (i, 0)


def _whole(i):
    return (0, 0)


def _tc2(accv, accd, newx, wih, whh, bih, bhh, wfc22, bfc22, w2attn, b21):
    full = lambda a: pl.BlockSpec(a.shape, _whole)
    return pl.pallas_call(
        _tc2_body,
        grid=(_NG,),
        in_specs=[
            pl.BlockSpec((_GB, F), _rows),
            pl.BlockSpec((_GB, F), lambda i: (i + _NG, 0)),
            pl.BlockSpec((_GB, F), _rows),
            pl.BlockSpec((_GB, F), lambda i: (i + _NG, 0)),
            pl.BlockSpec((_GB, F), _rows),
            full(wih), full(whh), full(bih), full(bhh),
            full(wfc22), full(bfc22), full(w2attn), full(b21),
        ],
        out_specs=(
            pl.BlockSpec((_GB, F), _rows),
            pl.BlockSpec((_GB, F), _rows),
            pl.BlockSpec((N // F, F), _whole),
            pl.BlockSpec((N // F, F), _whole),
        ),
        out_shape=(
            jax.ShapeDtypeStruct((N, F), jnp.float32),
            jax.ShapeDtypeStruct((N, F), jnp.float32),
            jax.ShapeDtypeStruct((N // F, F), jnp.float32),
            jax.ShapeDtypeStruct((N // F, F), jnp.float32),
        ),
    )(accv, accv, accd, accd, newx, wih, whh, bih, bhh,
      wfc22, bfc22, w2attn, b21)


def _tc3_body(accv_lo, accv_hi, accd_lo, accd_hi, out1, wih, whh, bih, bhh,
              lin, linb, out_o, allt_o, avg_o):
    t = (accv_lo[...] + accv_hi[...]) / (accd_lo[...] + accd_hi[...])
    ctx = _elu(t)
    o1 = out1[...]
    out2 = _gru(ctx, o1, wih[...], whh[...], bih[...], bhh[...])
    allt_o[0] = o1
    allt_o[1] = out2
    avg = (o1 + out2) * 0.5
    avg_o[...] = avg
    pre = _dott(avg, lin[...]) + linb[...]
    out_o[...] = _gelu(pre)


def _tc3(accv, accd, out1, wih, whh, bih, bhh, lin, linb):
    full = lambda a: pl.BlockSpec(a.shape, _whole)
    return pl.pallas_call(
        _tc3_body,
        grid=(_NG,),
        in_specs=[
            pl.BlockSpec((_GB, F), _rows),
            pl.BlockSpec((_GB, F), lambda i: (i + _NG, 0)),
            pl.BlockSpec((_GB, F), _rows),
            pl.BlockSpec((_GB, F), lambda i: (i + _NG, 0)),
            pl.BlockSpec((_GB, F), _rows),
            full(wih), full(whh), full(bih), full(bhh),
            full(lin), full(linb),
        ],
        out_specs=(
            pl.BlockSpec((_GB, F), _rows),
            pl.BlockSpec((2, _GB, F), lambda i: (0, i, 0)),
            pl.BlockSpec((_GB, F), _rows),
        ),
        out_shape=(
            jax.ShapeDtypeStruct((N, F), jnp.float32),
            jax.ShapeDtypeStruct((2, N, F), jnp.float32),
            jax.ShapeDtypeStruct((N, F), jnp.float32),
        ),
    )(accv, accv, accd, accd, out1, wih, whh, bih, bhh, lin, linb)


_MESH = plsc.VectorSubcoreMesh(core_axis_name="c", subcore_axis_name="s")
_SC_PARAMS = pltpu.CompilerParams(use_tc_tiling_on_sc=False,
                                  needs_layout_passes=False)


def _zero_shared(sbuf, accv_sh, accd_sh, sid):
    zv = jnp.zeros((16,), jnp.float32)

    @plsc.parallel_loop(0, CH, unroll=4)
    def _(r):
        for k in range(F // 16):
            sbuf[r, pl.ds(k * 16, 16)] = zv
    for h in range(ROWS_PER_TILE // CH):
        sl = pl.ds(sid * ROWS_PER_TILE + h * CH, CH)
        pltpu.sync_copy(sbuf, accv_sh.at[sl])
        pltpu.sync_copy(sbuf, accd_sh.at[sl])


def _scale_rows(gbuf, sbufv, sbufd, wbuf):
    @plsc.parallel_loop(0, CH, unroll=4)
    def _(r):
        wsp = plsc.load_gather(wbuf, [jnp.full((16,), r, jnp.int32)])
        for k in range(F // 16):
            sbufv[r, pl.ds(k * 16, 16)] = gbuf[r, pl.ds(k * 16, 16)] * wsp
            sbufd[r, pl.ds(k * 16, 16)] = wsp


def _emit_partials(accv_sh, accd_sh, outv_h, outd_h, cid, sid):
    sl = pl.ds(sid * ROWS_PER_TILE, ROWS_PER_TILE)
    osl = pl.ds(cid * N + sid * ROWS_PER_TILE, ROWS_PER_TILE)
    pltpu.sync_copy(accv_sh.at[sl], outv_h.at[osl])
    pltpu.sync_copy(accd_sh.at[sl], outd_h.at[osl])


def _sc1(tbla, tblb, et, alpha, a2, src2, dst2):
    @functools.partial(
        pl.kernel,
        mesh=_MESH,
        compiler_params=_SC_PARAMS,
        out_type=(
            jax.ShapeDtypeStruct((2 * N, F), jnp.float32),
            jax.ShapeDtypeStruct((2 * N, F), jnp.float32),
        ),
        scratch_types=[
            pltpu.VMEM((N // F, F), jnp.float32),
            pltpu.VMEM((1, F), jnp.float32),
            pltpu.VMEM((NCH, CH), jnp.int32),
            pltpu.VMEM((NCH, CH), jnp.int32),
            pltpu.VMEM((2, CH, F), jnp.float32),
            pltpu.VMEM((2, CH, F), jnp.float32),
            pltpu.VMEM((2, CH, F), jnp.float32),
            pltpu.VMEM((CH, F), jnp.float32),
            pltpu.VMEM((CH, F), jnp.float32),
            pltpu.VMEM((CH,), jnp.float32),
            pltpu.VMEM((CH, 17), jnp.float32),
            pltpu.VMEM_SHARED((N, F), jnp.float32),
            pltpu.VMEM_SHARED((N, F), jnp.float32),
            [pltpu.SemaphoreType.DMA] * 6,
        ],
    )
    def k(tbla_h, tblb_h, et_h, alpha_h, a2_h, src_h, dst_h, outv_h, outd_h,
          alpha_v, a2_v, src_v, dst_v, gbufa, gbufb, ebuf, sbufv, sbufd,
          wbuf, ptmp, accv_sh, accd_sh, sems):
        cid = lax.axis_index("c")
        sid = lax.axis_index("s")
        wid = cid * 16 + sid
        ebase = wid * EPT
        pre = [pltpu.async_copy(alpha_h, alpha_v, sems[0]),
               pltpu.async_copy(a2_h, a2_v, sems[1]),
               pltpu.async_copy(src_h.at[pl.ds(wid * NCH, NCH)], src_v, sems[2]),
               pltpu.async_copy(dst_h.at[pl.ds(wid * NCH, NCH)], dst_v, sems[3])]
        _zero_shared(sbufv, accv_sh, accd_sh, sid)
        for cp in pre:
            cp.wait()
        plsc.subcore_barrier()

        iot = lax.iota(jnp.int32, 16)
        zero16 = jnp.zeros((16,), jnp.float32)
        a2blk = [a2_v[0, pl.ds(k * 16, 16)] for k in range(F // 16)]

        def issue(c):
            s = c % 2
            cpa = pltpu.async_copy(tbla_h.at[src_v.at[c]], gbufa.at[s],
                                   sems[2 * s])
            cpb = pltpu.async_copy(tblb_h.at[src_v.at[c]], gbufb.at[s],
                                   sems[2 * s + 1])
            cpe = pltpu.async_copy(et_h.at[pl.ds(ebase + c * CH, CH)],
                                   ebuf.at[s], sems[4 + s])
            return cpa, cpb, cpe

        pend = issue(0)
        for c in range(NCH):
            s = c % 2
            for cp in pend:
                cp.wait()
            if c + 1 < NCH:
                pend = issue(c + 1)

            @plsc.parallel_loop(0, CH, unroll=4)
            def _(r):
                acc = zero16
                for k in range(F // 16):
                    a = gbufa[s, r, pl.ds(k * 16, 16)]
                    e = ebuf[s, r, pl.ds(k * 16, 16)]
                    acc = acc + _leaky(a + e) * a2blk[k]
                ptmp[r, pl.ds(0, 16)] = acc

            for g in range(CH // 16):
                rows = iot + g * 16
                tot = zero16
                for k in range(16):
                    tot = tot + plsc.load_gather(
                        ptmp, [rows, jnp.full((16,), k, jnp.int32)])
                dstv = dst_v[c, pl.ds(g * 16, 16)]
                ad = plsc.load_gather(
                    alpha_v,
                    [lax.shift_right_logical(dstv, 7), dstv & 127])
                wbuf[pl.ds(g * 16, 16)] = jnp.exp(_leaky(tot + ad))

            _scale_rows(gbufb.at[s], sbufv, sbufd, wbuf)
            pltpu.sync_copy(sbufv, accv_sh.at[dst_v.at[c]], add=True)
            pltpu.sync_copy(sbufd, accd_sh.at[dst_v.at[c]], add=True)

        plsc.subcore_barrier()
        _emit_partials(accv_sh, accd_sh, outv_h, outd_h, cid, sid)

    return k(tbla, tblb, et, alpha, a2, src2, dst2)


def _sc2(tbl2, b1, b2, src2, dst2):
    @functools.partial(
        pl.kernel,
        mesh=_MESH,
        compiler_params=_SC_PARAMS,
        out_type=(
            jax.ShapeDtypeStruct((2 * N, F), jnp.float32),
            jax.ShapeDtypeStruct((2 * N, F), jnp.float32),
        ),
        scratch_types=[
            pltpu.VMEM((N // F, F), jnp.float32),
            pltpu.VMEM((N // F, F), jnp.float32),
            pltpu.VMEM((NCH, CH), jnp.int32),
            pltpu.VMEM((NCH, CH), jnp.int32),
            pltpu.VMEM((2, CH, F), jnp.float32),
            pltpu.VMEM((CH, F), jnp.float32),
            pltpu.VMEM((CH, F), jnp.float32),
            pltpu.VMEM((CH,), jnp.float32),
            pltpu.VMEM_SHARED((N, F), jnp.float32),
            pltpu.VMEM_SHARED((N, F), jnp.float32),
            [pltpu.SemaphoreType.DMA] * 4,
        ],
    )
    def k(tbl_h, b1_h, b2_h, src_h, dst_h, outv_h, outd_h,
          b1_v, b2_v, src_v, dst_v, gbuf, sbufv, sbufd, wbuf,
          accv_sh, accd_sh, sems):
        cid = lax.axis_index("c")
        sid = lax.axis_index("s")
        wid = cid * 16 + sid
        pre = [pltpu.async_copy(b1_h, b1_v, sems[0]),
               pltpu.async_copy(b2_h, b2_v, sems[1]),
               pltpu.async_copy(src_h.at[pl.ds(wid * NCH, NCH)], src_v, sems[2]),
               pltpu.async_copy(dst_h.at[pl.ds(wid * NCH, NCH)], dst_v, sems[3])]
        _zero_shared(sbufv, accv_sh, accd_sh, sid)
        for cp in pre:
            cp.wait()
        plsc.subcore_barrier()

        def issue(c):
            s = c % 2
            return pltpu.async_copy(tbl_h.at[src_v.at[c]], gbuf.at[s], sems[s])

        pend = issue(0)
        for c in range(NCH):
            s = c % 2
            pend.wait()
            if c + 1 < NCH:
                pend = issue(c + 1)
            for g in range(CH // 16):
                dstv = dst_v[c, pl.ds(g * 16, 16)]
                srcv = src_v[c, pl.ds(g * 16, 16)]
                bd = plsc.load_gather(
                    b1_v, [lax.shift_right_logical(dstv, 7), dstv & 127])
                bs = plsc.load_gather(
                    b2_v, [lax.shift_right_logical(srcv, 7), srcv & 127])
                wbuf[pl.ds(g * 16, 16)] = jnp.exp(_leaky(bd + bs))
            _scale_rows(gbuf.at[s], sbufv, sbufd, wbuf)
            pltpu.sync_copy(sbufv, accv_sh.at[dst_v.at[c]], add=True)
            pltpu.sync_copy(sbufd, accd_sh.at[dst_v.at[c]], add=True)

        plsc.subcore_barrier()
        _emit_partials(accv_sh, accd_sh, outv_h, outd_h, cid, sid)

    return k(tbl2, b1, b2, src2, dst2)


def kernel(atom_features, edge_index, edge_attr,
           v1_fc1_w, v1_fc1_b, v1_fc2_w, v1_fc2_b,
           v1_attn_w, v1_attn_b, v1_attend_w, v1_attend_b,
           v1_gru_wih, v1_gru_whh, v1_gru_bih, v1_gru_bhh,
           v2_fc1_w, v2_fc1_b, v2_fc2_w, v2_fc2_b,
           v2_gru_wih, v2_gru_whh, v2_gru_bih, v2_gru_bhh,
           lin_w, lin_b):
    src2 = edge_index[0].reshape(E // CH, CH)
    dst2 = edge_index[1].reshape(E // CH, CH)

    newx, tbla, tblb, et, alpha, a2 = _tc1(
        atom_features, edge_attr.T,
        v1_fc1_w, v1_fc1_b.reshape(1, F),
        v1_fc2_w, v1_fc2_b.reshape(1, F),
        v1_attn_w, v1_attn_b.reshape(1, 1),
        v1_attend_w, v1_attend_b.reshape(1, F))

    accv1, accd1 = _sc1(tbla, tblb, et, alpha, a2, src2, dst2)

    out1, tbl2, b1, b2 = _tc2(
        accv1, accd1, newx,
        v1_gru_wih, v1_gru_whh,
        v1_gru_bih.reshape(1, 3 * F), v1_gru_bhh.reshape(1, 3 * F),
        v2_fc2_w, v2_fc2_b.reshape(1, F),
        v2_fc1_w, v2_fc1_b.reshape(1, 1))

    accv2, accd2 = _sc2(tbl2, b1, b2, src2, dst2)

    output, all_t, avg = _tc3(
        accv2, accd2, out1,
        v2_gru_wih, v2_gru_whh,
        v2_gru_bih.reshape(1, 3 * F), v2_gru_bhh.reshape(1, 3 * F),
        lin_w, lin_b.reshape(1, F))

    return (output, all_t, newx, avg)
